# Initial kernel scaffold; baseline (speedup 1.0000x reference)
import jax
import jax.numpy as jnp
from jax.experimental import pallas as pl


def kernel(x, edge_index, batch, convW1, convb1, bng1, bnb1, convW2, convb2, bns_g, bns_b, fcW, fcb):
    raise NotImplementedError("write your pallas kernel here")


# speedup vs baseline: 2.7478x; 2.7478x over previous
import functools

import jax
import jax.numpy as jnp
from jax import lax
from jax.experimental import pallas as pl
from jax.experimental.pallas import tpu as pltpu
from jax.experimental.pallas import tpu_sc as plsc

N = 10000
E = 320000
D = 128
C = 40
L = 3
G = 128

NC = 2
NS = 16
NW = NC * NS
K = 128
CHUNKS = 80
EPW = CHUNKS * K
E_PAD = NW * EPW
N_ACC = 10240
RPT = N_ACC // NS

ROW_BLK = 400
N_BLKS = N // ROW_BLK


def _sc_body(src_hbm, dst_hbm, zeros_hbm, h_hbm, out_hbm,
             sidx, didx, rows, acc, sem):
    cid = lax.axis_index("c")
    sid = lax.axis_index("s")
    wid = cid * NS + sid
    pltpu.sync_copy(zeros_hbm, acc.at[pl.ds(sid * RPT, RPT)])
    pltpu.sync_copy(src_hbm.at[wid], sidx)
    pltpu.sync_copy(dst_hbm.at[wid], didx)
    plsc.subcore_barrier()

    def eloop(j, c):
        pltpu.async_copy(h_hbm.at[sidx.at[j]], rows, sem).wait()
        pltpu.sync_copy(rows, acc.at[didx.at[j]], add=True)
        return c

    lax.fori_loop(0, CHUNKS, eloop, 0)
    plsc.subcore_barrier()
    pltpu.sync_copy(acc.at[pl.ds(sid * RPT, RPT)],
                    out_hbm.at[pl.ds(cid * N_ACC + sid * RPT, RPT)])


_sc_seg_sum = functools.partial(
    pl.kernel,
    mesh=plsc.VectorSubcoreMesh(core_axis_name="c", subcore_axis_name="s"),
    out_type=jax.ShapeDtypeStruct((NC * N_ACC, D), jnp.float32),
    scratch_types=[
        pltpu.VMEM((CHUNKS, K), jnp.int32),
        pltpu.VMEM((CHUNKS, K), jnp.int32),
        pltpu.VMEM((K, D), jnp.float32),
        pltpu.VMEM_SHARED((N_ACC, D), jnp.float32),
        pltpu.SemaphoreType.DMA,
    ],
)(_sc_body)


def _layer_body(h_ref, p0_ref, p1_ref, w1_ref, w2_ref,
                s1_ref, t1_ref, s2_ref, t2_ref, o_ref):
    z = h_ref[...] + p0_ref[0] + p1_ref[0]
    a = jnp.dot(z, w1_ref[...], preferred_element_type=jnp.float32)
    a = jnp.maximum(a * s1_ref[...] + t1_ref[...], 0.0)
    b = jnp.dot(a, w2_ref[...], preferred_element_type=jnp.float32)
    o_ref[...] = jnp.maximum(b * s2_ref[...] + t2_ref[...], 0.0)


def _layer_mlp(h, parts, w1, w2, s1, t1, s2, t2):
    full = lambda i: (0, 0)
    return pl.pallas_call(
        _layer_body,
        grid=(N_BLKS,),
        in_specs=[
            pl.BlockSpec((ROW_BLK, D), lambda i: (i, 0)),
            pl.BlockSpec((1, ROW_BLK, D), lambda i: (0, i, 0)),
            pl.BlockSpec((1, ROW_BLK, D), lambda i: (1, i, 0)),
            pl.BlockSpec((D, D), full),
            pl.BlockSpec((D, D), full),
            pl.BlockSpec((1, D), full),
            pl.BlockSpec((1, D), full),
            pl.BlockSpec((1, D), full),
            pl.BlockSpec((1, D), full),
        ],
        out_specs=pl.BlockSpec((ROW_BLK, D), lambda i: (i, 0)),
        out_shape=jax.ShapeDtypeStruct((N, D), jnp.float32),
    )(h, parts, parts, w1, w2, s1, t1, s2, t2)


def _pool_body(x_ref, h1_ref, h2_ref, h3_ref, b_ref, fw_ref, sb_ref,
               o_ref, acc_ref):
    i = pl.program_id(0)

    @pl.when(i == 0)
    def _():
        acc_ref[...] = jnp.zeros_like(acc_ref)

    bids = b_ref[0, 0, :]
    m = (lax.broadcasted_iota(jnp.int32, (G, ROW_BLK), 0)
         == bids[None, :]).astype(jnp.float32)
    for k, o in enumerate((x_ref, h1_ref, h2_ref, h3_ref)):
        acc_ref[k] = acc_ref[k] + jnp.dot(
            m, o[...], preferred_element_type=jnp.float32)

    @pl.when(i == N_BLKS - 1)
    def _():
        v = sb_ref[...]
        for k in range(L + 1):
            v = v + jnp.dot(acc_ref[k], fw_ref[k],
                            preferred_element_type=jnp.float32)
        cols = lax.broadcasted_iota(jnp.int32, (G, D), 1)
        valid = cols < C
        vm = jnp.where(valid, v, -1e30)
        mx = jnp.max(vm, axis=1, keepdims=True)
        ex = jnp.where(valid, jnp.exp(vm - mx), 0.0)
        s = jnp.sum(ex, axis=1, keepdims=True)
        o_ref[...] = vm - mx - jnp.log(s)


def _pool_heads(x, h1, h2, h3, batch3, fw_pad, sumb):
    blk = lambda i: (i, 0)
    full = lambda i: (0, 0)
    return pl.pallas_call(
        _pool_body,
        grid=(N_BLKS,),
        in_specs=[
            pl.BlockSpec((ROW_BLK, D), blk),
            pl.BlockSpec((ROW_BLK, D), blk),
            pl.BlockSpec((ROW_BLK, D), blk),
            pl.BlockSpec((ROW_BLK, D), blk),
            pl.BlockSpec((1, 1, ROW_BLK), lambda i: (i, 0, 0)),
            pl.BlockSpec((L + 1, D, D), lambda i: (0, 0, 0)),
            pl.BlockSpec((1, D), full),
        ],
        out_specs=pl.BlockSpec((G, D), full),
        out_shape=jax.ShapeDtypeStruct((G, D), jnp.float32),
        scratch_shapes=[pltpu.VMEM((L + 1, G, D), jnp.float32)],
    )(x, h1, h2, h3, batch3, fw_pad, sumb)


def kernel(x, edge_index, batch, convW1, convb1, bng1, bnb1,
           convW2, convb2, bns_g, bns_b, fcW, fcb):
    src = edge_index[0]
    dst = edge_index[1]
    pad = E_PAD - E
    src3 = jnp.concatenate(
        [src, jnp.zeros((pad,), jnp.int32)]).reshape(NW, CHUNKS, K)
    dst3 = jnp.concatenate(
        [dst, jnp.full((pad,), N, jnp.int32)]).reshape(NW, CHUNKS, K)
    zeros_blk = jnp.zeros((RPT, D), jnp.float32)

    s1 = bng1.reshape(L, 1, D)
    t1 = (convb1 * bng1 + bnb1).reshape(L, 1, D)
    s2 = bns_g.reshape(L, 1, D)
    t2 = (convb2 * bns_g + bns_b).reshape(L, 1, D)

    outs = [x]
    h = x
    for i in range(L):
        parts = _sc_seg_sum(src3, dst3, zeros_blk, h).reshape(NC, N_ACC, D)
        h = _layer_mlp(h, parts, convW1[i], convW2[i],
                       s1[i], t1[i], s2[i], t2[i])
        outs.append(h)

    batch3 = batch.reshape(N_BLKS, 1, ROW_BLK)
    fw_pad = jnp.concatenate(
        [fcW, jnp.zeros((L + 1, D, D - C), jnp.float32)], axis=2)
    sumb = jnp.concatenate(
        [jnp.sum(fcb, axis=0), jnp.zeros((D - C,), jnp.float32)]
    ).reshape(1, D)

    out = _pool_heads(outs[0], outs[1], outs[2], outs[3],
                      batch3, fw_pad, sumb)
    return out[:, :C]


# speedup vs baseline: 3.0392x; 1.1060x over previous
import functools

import jax
import jax.numpy as jnp
from jax import lax
from jax.experimental import pallas as pl
from jax.experimental.pallas import tpu as pltpu
from jax.experimental.pallas import tpu_sc as plsc

N = 10000
E = 320000
D = 128
C = 40
L = 3
G = 128

NC = 2
NS = 16
NW = NC * NS
K = 80
CHUNKS = 128
EPW = CHUNKS * K
E_PAD = NW * EPW
N_ACC = 10240
RPT = N_ACC // NS

ROW_BLK = 400
N_BLKS = N // ROW_BLK


NBUF = 4
ROUNDS = CHUNKS // NBUF


def _sc_body(src_hbm, dst_hbm, zeros_hbm, h_hbm, out_hbm, *sc):
    sidx = sc[0:NBUF]
    didx = sc[NBUF:2 * NBUF]
    rows = sc[2 * NBUF:3 * NBUF]
    acc = sc[3 * NBUF]
    isem = sc[3 * NBUF + 1:4 * NBUF + 1]
    dsem = sc[4 * NBUF + 1:5 * NBUF + 1]
    gsem = sc[5 * NBUF + 1:6 * NBUF + 1]
    ssem = sc[6 * NBUF + 1:7 * NBUF + 1]
    cid = lax.axis_index("c")
    sid = lax.axis_index("s")
    wid = cid * NS + sid
    pltpu.sync_copy(zeros_hbm, acc.at[pl.ds(sid * RPT, RPT)])
    plsc.subcore_barrier()

    for b in range(NBUF):
        pltpu.async_copy(src_hbm.at[wid, b], sidx[b], isem[b])
        pltpu.async_copy(dst_hbm.at[wid, b], didx[b], dsem[b])
    for b in range(NBUF):
        pltpu.make_async_copy(src_hbm.at[wid, b], sidx[b], isem[b]).wait()
        pltpu.async_copy(h_hbm.at[sidx[b]], rows[b], gsem[b])

    def eloop(i, c):
        for b in range(NBUF):
            j = i * NBUF + b
            pltpu.make_async_copy(h_hbm.at[sidx[b]], rows[b], gsem[b]).wait()
            pltpu.make_async_copy(
                dst_hbm.at[wid, j], didx[b], dsem[b]).wait()
            pltpu.async_copy(rows[b], acc.at[didx[b]], ssem[b], add=True)
        @pl.when(i < ROUNDS - 1)
        def _():
            for b in range(NBUF):
                jn = i * NBUF + b + NBUF
                pltpu.async_copy(src_hbm.at[wid, jn], sidx[b], isem[b])
                pltpu.make_async_copy(
                    rows[b], acc.at[didx[b]], ssem[b]).wait()
                pltpu.async_copy(dst_hbm.at[wid, jn], didx[b], dsem[b])
                pltpu.make_async_copy(
                    src_hbm.at[wid, jn], sidx[b], isem[b]).wait()
                pltpu.async_copy(h_hbm.at[sidx[b]], rows[b], gsem[b])
        return c

    lax.fori_loop(0, ROUNDS, eloop, 0)
    for b in range(NBUF):
        pltpu.make_async_copy(rows[b], acc.at[didx[b]], ssem[b]).wait()
    plsc.subcore_barrier()
    pltpu.sync_copy(acc.at[pl.ds(sid * RPT, RPT)],
                    out_hbm.at[pl.ds(cid * N_ACC + sid * RPT, RPT)])


_sc_seg_sum = functools.partial(
    pl.kernel,
    mesh=plsc.VectorSubcoreMesh(core_axis_name="c", subcore_axis_name="s"),
    out_type=jax.ShapeDtypeStruct((NC * N_ACC, D), jnp.float32),
    scratch_types=(
        [pltpu.VMEM((K,), jnp.int32) for _ in range(2 * NBUF)]
        + [pltpu.VMEM((K, D), jnp.float32) for _ in range(NBUF)]
        + [pltpu.VMEM_SHARED((N_ACC, D), jnp.float32)]
        + [pltpu.SemaphoreType.DMA for _ in range(4 * NBUF)]
    ),
)(_sc_body)


def _layer_body(h_ref, p0_ref, p1_ref, w1_ref, w2_ref,
                s1_ref, t1_ref, s2_ref, t2_ref, o_ref):
    z = h_ref[...] + p0_ref[0] + p1_ref[0]
    a = jnp.dot(z, w1_ref[...], preferred_element_type=jnp.float32)
    a = jnp.maximum(a * s1_ref[...] + t1_ref[...], 0.0)
    b = jnp.dot(a, w2_ref[...], preferred_element_type=jnp.float32)
    o_ref[...] = jnp.maximum(b * s2_ref[...] + t2_ref[...], 0.0)


def _layer_mlp(h, parts, w1, w2, s1, t1, s2, t2):
    full = lambda i: (0, 0)
    return pl.pallas_call(
        _layer_body,
        grid=(N_BLKS,),
        in_specs=[
            pl.BlockSpec((ROW_BLK, D), lambda i: (i, 0)),
            pl.BlockSpec((1, ROW_BLK, D), lambda i: (0, i, 0)),
            pl.BlockSpec((1, ROW_BLK, D), lambda i: (1, i, 0)),
            pl.BlockSpec((D, D), full),
            pl.BlockSpec((D, D), full),
            pl.BlockSpec((1, D), full),
            pl.BlockSpec((1, D), full),
            pl.BlockSpec((1, D), full),
            pl.BlockSpec((1, D), full),
        ],
        out_specs=pl.BlockSpec((ROW_BLK, D), lambda i: (i, 0)),
        out_shape=jax.ShapeDtypeStruct((N, D), jnp.float32),
    )(h, parts, parts, w1, w2, s1, t1, s2, t2)


def _pool_body(x_ref, h1_ref, h2_ref, h3_ref, b_ref, fw_ref, sb_ref,
               o_ref, acc_ref):
    i = pl.program_id(0)

    @pl.when(i == 0)
    def _():
        acc_ref[...] = jnp.zeros_like(acc_ref)

    bids = b_ref[0, 0, :]
    m = (lax.broadcasted_iota(jnp.int32, (G, ROW_BLK), 0)
         == bids[None, :]).astype(jnp.float32)
    for k, o in enumerate((x_ref, h1_ref, h2_ref, h3_ref)):
        acc_ref[k] = acc_ref[k] + jnp.dot(
            m, o[...], preferred_element_type=jnp.float32)

    @pl.when(i == N_BLKS - 1)
    def _():
        v = sb_ref[...]
        for k in range(L + 1):
            v = v + jnp.dot(acc_ref[k], fw_ref[k],
                            preferred_element_type=jnp.float32)
        cols = lax.broadcasted_iota(jnp.int32, (G, D), 1)
        valid = cols < C
        vm = jnp.where(valid, v, -1e30)
        mx = jnp.max(vm, axis=1, keepdims=True)
        ex = jnp.where(valid, jnp.exp(vm - mx), 0.0)
        s = jnp.sum(ex, axis=1, keepdims=True)
        o_ref[...] = vm - mx - jnp.log(s)


def _pool_heads(x, h1, h2, h3, batch3, fw_pad, sumb):
    blk = lambda i: (i, 0)
    full = lambda i: (0, 0)
    return pl.pallas_call(
        _pool_body,
        grid=(N_BLKS,),
        in_specs=[
            pl.BlockSpec((ROW_BLK, D), blk),
            pl.BlockSpec((ROW_BLK, D), blk),
            pl.BlockSpec((ROW_BLK, D), blk),
            pl.BlockSpec((ROW_BLK, D), blk),
            pl.BlockSpec((1, 1, ROW_BLK), lambda i: (i, 0, 0)),
            pl.BlockSpec((L + 1, D, D), lambda i: (0, 0, 0)),
            pl.BlockSpec((1, D), full),
        ],
        out_specs=pl.BlockSpec((G, D), full),
        out_shape=jax.ShapeDtypeStruct((G, D), jnp.float32),
        scratch_shapes=[pltpu.VMEM((L + 1, G, D), jnp.float32)],
    )(x, h1, h2, h3, batch3, fw_pad, sumb)


def kernel(x, edge_index, batch, convW1, convb1, bng1, bnb1,
           convW2, convb2, bns_g, bns_b, fcW, fcb):
    src = edge_index[0]
    dst = edge_index[1]
    pad = E_PAD - E
    src3 = jnp.concatenate(
        [src, jnp.zeros((pad,), jnp.int32)]).reshape(NW, CHUNKS, K)
    dst3 = jnp.concatenate(
        [dst, jnp.full((pad,), N, jnp.int32)]).reshape(NW, CHUNKS, K)
    zeros_blk = jnp.zeros((RPT, D), jnp.float32)

    s1 = bng1.reshape(L, 1, D)
    t1 = (convb1 * bng1 + bnb1).reshape(L, 1, D)
    s2 = bns_g.reshape(L, 1, D)
    t2 = (convb2 * bns_g + bns_b).reshape(L, 1, D)

    outs = [x]
    h = x
    for i in range(L):
        parts = _sc_seg_sum(src3, dst3, zeros_blk, h).reshape(NC, N_ACC, D)
        h = _layer_mlp(h, parts, convW1[i], convW2[i],
                       s1[i], t1[i], s2[i], t2[i])
        outs.append(h)

    batch3 = batch.reshape(N_BLKS, 1, ROW_BLK)
    fw_pad = jnp.concatenate(
        [fcW, jnp.zeros((L + 1, D, D - C), jnp.float32)], axis=2)
    sumb = jnp.concatenate(
        [jnp.sum(fcb, axis=0), jnp.zeros((D - C,), jnp.float32)]
    ).reshape(1, D)

    out = _pool_heads(outs[0], outs[1], outs[2], outs[3],
                      batch3, fw_pad, sumb)
    return out[:, :C]


# speedup vs baseline: 5.2577x; 1.7300x over previous
import functools

import jax
import jax.numpy as jnp
from jax import lax
from jax.experimental import pallas as pl
from jax.experimental.pallas import tpu as pltpu
from jax.experimental.pallas import tpu_sc as plsc

N = 10000
E = 320000
D = 128
C = 40
L = 3
G = 128

NC = 2
NS = 16
NW = NC * NS
K = 80
CHUNKS = 128
EPW = CHUNKS * K
E_PAD = NW * EPW
N_ACC = 10240
RPT = N_ACC // NS

ROW_BLK = 400
N_BLKS = N // ROW_BLK


NBUF = 4
ROUNDS = CHUNKS // NBUF


def _sc_body(src_hbm, dst_hbm, zeros_hbm, h_hbm, out_hbm, *sc):
    sidx = sc[0:NBUF]
    didx = sc[NBUF:2 * NBUF]
    rows = sc[2 * NBUF:3 * NBUF]
    acc = sc[3 * NBUF]
    isem = sc[3 * NBUF + 1:4 * NBUF + 1]
    dsem = sc[4 * NBUF + 1:5 * NBUF + 1]
    gsem = sc[5 * NBUF + 1:6 * NBUF + 1]
    ssem = sc[6 * NBUF + 1:7 * NBUF + 1]
    cid = lax.axis_index("c")
    sid = lax.axis_index("s")
    wid = cid * NS + sid
    pltpu.sync_copy(zeros_hbm, acc.at[pl.ds(sid * RPT, RPT)])
    plsc.subcore_barrier()

    for b in range(NBUF):
        pltpu.async_copy(src_hbm.at[wid, b], sidx[b], isem[b])
        pltpu.async_copy(dst_hbm.at[wid, b], didx[b], dsem[b])
    for b in range(NBUF):
        pltpu.make_async_copy(src_hbm.at[wid, b], sidx[b], isem[b]).wait()
        pltpu.async_copy(h_hbm.at[sidx[b]], rows[b], gsem[b])

    def eloop(i, c):
        for b in range(NBUF):
            j = i * NBUF + b
            pltpu.make_async_copy(h_hbm.at[sidx[b]], rows[b], gsem[b]).wait()
            pltpu.make_async_copy(
                dst_hbm.at[wid, j], didx[b], dsem[b]).wait()
            pltpu.async_copy(rows[b], acc.at[didx[b]], ssem[b], add=True)
        @pl.when(i < ROUNDS - 1)
        def _():
            for b in range(NBUF):
                jn = i * NBUF + b + NBUF
                pltpu.async_copy(src_hbm.at[wid, jn], sidx[b], isem[b])
                pltpu.make_async_copy(
                    rows[b], acc.at[didx[b]], ssem[b]).wait()
                pltpu.async_copy(dst_hbm.at[wid, jn], didx[b], dsem[b])
                pltpu.make_async_copy(
                    src_hbm.at[wid, jn], sidx[b], isem[b]).wait()
                pltpu.async_copy(h_hbm.at[sidx[b]], rows[b], gsem[b])
        return c

    lax.fori_loop(0, ROUNDS, eloop, 0)
    for b in range(NBUF):
        pltpu.make_async_copy(rows[b], acc.at[didx[b]], ssem[b]).wait()
    plsc.subcore_barrier()
    pltpu.sync_copy(acc.at[pl.ds(sid * RPT, RPT)],
                    out_hbm.at[pl.ds(cid * N_ACC + sid * RPT, RPT)])


_sc_seg_sum = functools.partial(
    pl.kernel,
    mesh=plsc.VectorSubcoreMesh(core_axis_name="c", subcore_axis_name="s"),
    out_type=jax.ShapeDtypeStruct((NC * N_ACC, D), jnp.float32),
    scratch_types=(
        [pltpu.VMEM((K,), jnp.int32) for _ in range(2 * NBUF)]
        + [pltpu.VMEM((K, D), jnp.float32) for _ in range(NBUF)]
        + [pltpu.VMEM_SHARED((N_ACC, D), jnp.float32)]
        + [pltpu.SemaphoreType.DMA for _ in range(4 * NBUF)]
    ),
)(_sc_body)


def _layer_body(h_ref, p0_ref, p1_ref, w1_ref, w2_ref,
                s1_ref, t1_ref, s2_ref, t2_ref, o_ref):
    z = h_ref[...] + p0_ref[0] + p1_ref[0]
    a = jnp.dot(z, w1_ref[...], preferred_element_type=jnp.float32)
    a = jnp.maximum(a * s1_ref[...] + t1_ref[...], 0.0)
    b = jnp.dot(a, w2_ref[...], preferred_element_type=jnp.float32)
    o_ref[...] = jnp.maximum(b * s2_ref[...] + t2_ref[...], 0.0)


def _layer_mlp(h, parts, w1, w2, s1, t1, s2, t2):
    full = lambda i: (0, 0)
    return pl.pallas_call(
        _layer_body,
        grid=(N_BLKS,),
        in_specs=[
            pl.BlockSpec((ROW_BLK, D), lambda i: (i, 0)),
            pl.BlockSpec((1, ROW_BLK, D), lambda i: (0, i, 0)),
            pl.BlockSpec((1, ROW_BLK, D), lambda i: (1, i, 0)),
            pl.BlockSpec((D, D), full),
            pl.BlockSpec((D, D), full),
            pl.BlockSpec((1, D), full),
            pl.BlockSpec((1, D), full),
            pl.BlockSpec((1, D), full),
            pl.BlockSpec((1, D), full),
        ],
        out_specs=pl.BlockSpec((ROW_BLK, D), lambda i: (i, 0)),
        out_shape=jax.ShapeDtypeStruct((N, D), jnp.float32),
    )(h, parts, parts, w1, w2, s1, t1, s2, t2)


def _pool_body(x_ref, h1_ref, h2_ref, h3_ref, b_ref, fw_ref, sb_ref,
               o_ref, acc_ref):
    i = pl.program_id(0)

    @pl.when(i == 0)
    def _():
        acc_ref[...] = jnp.zeros_like(acc_ref)

    bids = b_ref[0, 0, :]
    m = (lax.broadcasted_iota(jnp.int32, (G, ROW_BLK), 0)
         == bids[None, :]).astype(jnp.float32)
    for k, o in enumerate((x_ref, h1_ref, h2_ref, h3_ref)):
        acc_ref[k] = acc_ref[k] + jnp.dot(
            m, o[...], preferred_element_type=jnp.float32)

    @pl.when(i == N_BLKS - 1)
    def _():
        v = sb_ref[...]
        for k in range(L + 1):
            v = v + jnp.dot(acc_ref[k], fw_ref[k],
                            preferred_element_type=jnp.float32)
        cols = lax.broadcasted_iota(jnp.int32, (G, D), 1)
        valid = cols < C
        vm = jnp.where(valid, v, -1e30)
        mx = jnp.max(vm, axis=1, keepdims=True)
        ex = jnp.where(valid, jnp.exp(vm - mx), 0.0)
        s = jnp.sum(ex, axis=1, keepdims=True)
        o_ref[...] = vm - mx - jnp.log(s)


def _pool_heads(x, h1, h2, h3, batch3, fw_pad, sumb):
    blk = lambda i: (i, 0)
    full = lambda i: (0, 0)
    return pl.pallas_call(
        _pool_body,
        grid=(N_BLKS,),
        in_specs=[
            pl.BlockSpec((ROW_BLK, D), blk),
            pl.BlockSpec((ROW_BLK, D), blk),
            pl.BlockSpec((ROW_BLK, D), blk),
            pl.BlockSpec((ROW_BLK, D), blk),
            pl.BlockSpec((1, 1, ROW_BLK), lambda i: (i, 0, 0)),
            pl.BlockSpec((L + 1, D, D), lambda i: (0, 0, 0)),
            pl.BlockSpec((1, D), full),
        ],
        out_specs=pl.BlockSpec((G, D), full),
        out_shape=jax.ShapeDtypeStruct((G, D), jnp.float32),
        scratch_shapes=[pltpu.VMEM((L + 1, G, D), jnp.float32)],
    )(x, h1, h2, h3, batch3, fw_pad, sumb)


def kernel(x, edge_index, batch, convW1, convb1, bng1, bnb1,
           convW2, convb2, bns_g, bns_b, fcW, fcb):
    src = edge_index[0]
    dst = edge_index[1]
    pad = E_PAD - E
    order = jnp.argsort(dst)
    pad_ar = lax.iota(jnp.int32, pad)
    src3 = jnp.concatenate(
        [src[order], pad_ar % N]).reshape(NW, CHUNKS, K)
    dst3 = jnp.concatenate(
        [dst[order], N + pad_ar % (N_ACC - N)]).reshape(NW, CHUNKS, K)
    zeros_blk = jnp.zeros((RPT, D), jnp.float32)

    s1 = bng1.reshape(L, 1, D)
    t1 = (convb1 * bng1 + bnb1).reshape(L, 1, D)
    s2 = bns_g.reshape(L, 1, D)
    t2 = (convb2 * bns_g + bns_b).reshape(L, 1, D)

    outs = [x]
    h = x
    for i in range(L):
        parts = _sc_seg_sum(src3, dst3, zeros_blk, h).reshape(NC, N_ACC, D)
        h = _layer_mlp(h, parts, convW1[i], convW2[i],
                       s1[i], t1[i], s2[i], t2[i])
        outs.append(h)

    batch3 = batch.reshape(N_BLKS, 1, ROW_BLK)
    fw_pad = jnp.concatenate(
        [fcW, jnp.zeros((L + 1, D, D - C), jnp.float32)], axis=2)
    sumb = jnp.concatenate(
        [jnp.sum(fcb, axis=0), jnp.zeros((D - C,), jnp.float32)]
    ).reshape(1, D)

    out = _pool_heads(outs[0], outs[1], outs[2], outs[3],
                      batch3, fw_pad, sumb)
    return out[:, :C]


# speedup vs baseline: 6.2725x; 1.1930x over previous
import functools

import jax
import jax.numpy as jnp
from jax import lax
from jax.experimental import pallas as pl
from jax.experimental.pallas import tpu as pltpu
from jax.experimental.pallas import tpu_sc as plsc

N = 10000
E = 320000
D = 128
C = 40
L = 3
G = 128

NC = 2
NS = 16
NW = NC * NS
K = 80
CHUNKS = 128
EPW = CHUNKS * K
E_PAD = NW * EPW
N_ACC = 10240
RPT = N_ACC // NS

ROW_BLK = 400
N_BLKS = N // ROW_BLK


NBUF = 4
ROUNDS = CHUNKS // NBUF


def _sc_body(src_hbm, dst_hbm, zeros_hbm, h_hbm, out_hbm, *sc):
    sidx = sc[0:NBUF]
    didx = sc[NBUF:2 * NBUF]
    rows = sc[2 * NBUF:3 * NBUF]
    acc = sc[3 * NBUF]
    isem = sc[3 * NBUF + 1:4 * NBUF + 1]
    dsem = sc[4 * NBUF + 1:5 * NBUF + 1]
    gsem = sc[5 * NBUF + 1:6 * NBUF + 1]
    ssem = sc[6 * NBUF + 1:7 * NBUF + 1]
    cid = lax.axis_index("c")
    sid = lax.axis_index("s")
    wid = cid * NS + sid
    pltpu.sync_copy(zeros_hbm, acc.at[pl.ds(sid * RPT, RPT)])
    plsc.subcore_barrier()

    for b in range(NBUF):
        pltpu.async_copy(src_hbm.at[wid, b], sidx[b], isem[b])
        pltpu.async_copy(dst_hbm.at[wid, b], didx[b], dsem[b])
    for b in range(NBUF):
        pltpu.make_async_copy(src_hbm.at[wid, b], sidx[b], isem[b]).wait()
        pltpu.async_copy(h_hbm.at[sidx[b]], rows[b], gsem[b])

    def eloop(i, c):
        for b in range(NBUF):
            j = i * NBUF + b
            pltpu.make_async_copy(h_hbm.at[sidx[b]], rows[b], gsem[b]).wait()
            pltpu.make_async_copy(
                dst_hbm.at[wid, j], didx[b], dsem[b]).wait()
            pltpu.async_copy(rows[b], acc.at[didx[b]], ssem[b], add=True)
        @pl.when(i < ROUNDS - 1)
        def _():
            for b in range(NBUF):
                jn = i * NBUF + b + NBUF
                pltpu.async_copy(src_hbm.at[wid, jn], sidx[b], isem[b])
                pltpu.make_async_copy(
                    rows[b], acc.at[didx[b]], ssem[b]).wait()
                pltpu.async_copy(dst_hbm.at[wid, jn], didx[b], dsem[b])
                pltpu.make_async_copy(
                    src_hbm.at[wid, jn], sidx[b], isem[b]).wait()
                pltpu.async_copy(h_hbm.at[sidx[b]], rows[b], gsem[b])
        return c

    lax.fori_loop(0, ROUNDS, eloop, 0)
    for b in range(NBUF):
        pltpu.make_async_copy(rows[b], acc.at[didx[b]], ssem[b]).wait()
    plsc.subcore_barrier()
    pltpu.sync_copy(acc.at[pl.ds(sid * RPT, RPT)],
                    out_hbm.at[pl.ds(cid * N_ACC + sid * RPT, RPT)])


_sc_seg_sum = functools.partial(
    pl.kernel,
    mesh=plsc.VectorSubcoreMesh(core_axis_name="c", subcore_axis_name="s"),
    out_type=jax.ShapeDtypeStruct((NC * N_ACC, D), jnp.float32),
    scratch_types=(
        [pltpu.VMEM((K,), jnp.int32) for _ in range(2 * NBUF)]
        + [pltpu.VMEM((K, D), jnp.float32) for _ in range(NBUF)]
        + [pltpu.VMEM_SHARED((N_ACC, D), jnp.float32)]
        + [pltpu.SemaphoreType.DMA for _ in range(4 * NBUF)]
    ),
)(_sc_body)


def _layer_body(h_ref, p0_ref, p1_ref, w1_ref, w2_ref,
                s1_ref, t1_ref, s2_ref, t2_ref, o_ref):
    z = h_ref[...] + p0_ref[0] + p1_ref[0]
    a = jnp.dot(z, w1_ref[...], preferred_element_type=jnp.float32)
    a = jnp.maximum(a * s1_ref[...] + t1_ref[...], 0.0)
    b = jnp.dot(a, w2_ref[...], preferred_element_type=jnp.float32)
    o_ref[...] = jnp.maximum(b * s2_ref[...] + t2_ref[...], 0.0)


def _layer_mlp(h, parts, w1, w2, s1, t1, s2, t2):
    full = lambda i: (0, 0)
    return pl.pallas_call(
        _layer_body,
        grid=(N_BLKS,),
        in_specs=[
            pl.BlockSpec((ROW_BLK, D), lambda i: (i, 0)),
            pl.BlockSpec((1, ROW_BLK, D), lambda i: (0, i, 0)),
            pl.BlockSpec((1, ROW_BLK, D), lambda i: (1, i, 0)),
            pl.BlockSpec((D, D), full),
            pl.BlockSpec((D, D), full),
            pl.BlockSpec((1, D), full),
            pl.BlockSpec((1, D), full),
            pl.BlockSpec((1, D), full),
            pl.BlockSpec((1, D), full),
        ],
        out_specs=pl.BlockSpec((ROW_BLK, D), lambda i: (i, 0)),
        out_shape=jax.ShapeDtypeStruct((N, D), jnp.float32),
    )(h, parts, parts, w1, w2, s1, t1, s2, t2)


def _pool_body(x_ref, h1_ref, h2_ref, h3_ref, b_ref, fw_ref, sb_ref,
               o_ref, acc_ref):
    i = pl.program_id(0)

    @pl.when(i == 0)
    def _():
        acc_ref[...] = jnp.zeros_like(acc_ref)

    bids = b_ref[0, 0, :]
    m = (lax.broadcasted_iota(jnp.int32, (G, ROW_BLK), 0)
         == bids[None, :]).astype(jnp.float32)
    for k, o in enumerate((x_ref, h1_ref, h2_ref, h3_ref)):
        acc_ref[k] = acc_ref[k] + jnp.dot(
            m, o[...], preferred_element_type=jnp.float32)

    @pl.when(i == N_BLKS - 1)
    def _():
        v = sb_ref[...]
        for k in range(L + 1):
            v = v + jnp.dot(acc_ref[k], fw_ref[k],
                            preferred_element_type=jnp.float32)
        cols = lax.broadcasted_iota(jnp.int32, (G, D), 1)
        valid = cols < C
        vm = jnp.where(valid, v, -1e30)
        mx = jnp.max(vm, axis=1, keepdims=True)
        ex = jnp.where(valid, jnp.exp(vm - mx), 0.0)
        s = jnp.sum(ex, axis=1, keepdims=True)
        o_ref[...] = vm - mx - jnp.log(s)


def _pool_heads(x, h1, h2, h3, batch3, fw_pad, sumb):
    blk = lambda i: (i, 0)
    full = lambda i: (0, 0)
    return pl.pallas_call(
        _pool_body,
        grid=(N_BLKS,),
        in_specs=[
            pl.BlockSpec((ROW_BLK, D), blk),
            pl.BlockSpec((ROW_BLK, D), blk),
            pl.BlockSpec((ROW_BLK, D), blk),
            pl.BlockSpec((ROW_BLK, D), blk),
            pl.BlockSpec((1, 1, ROW_BLK), lambda i: (i, 0, 0)),
            pl.BlockSpec((L + 1, D, D), lambda i: (0, 0, 0)),
            pl.BlockSpec((1, D), full),
        ],
        out_specs=pl.BlockSpec((G, D), full),
        out_shape=jax.ShapeDtypeStruct((G, D), jnp.float32),
        scratch_shapes=[pltpu.VMEM((L + 1, G, D), jnp.float32)],
    )(x, h1, h2, h3, batch3, fw_pad, sumb)


def kernel(x, edge_index, batch, convW1, convb1, bng1, bnb1,
           convW2, convb2, bns_g, bns_b, fcW, fcb):
    src = edge_index[0]
    dst = edge_index[1]
    padw = (E_PAD - E) // NW
    key = (dst << 14) | src
    pad_ar = lax.iota(jnp.int32, padw)
    padk = ((N + pad_ar) << 14) | (pad_ar * 41 % N)
    allk = jnp.concatenate(
        [key.reshape(NW, E // NW),
         jnp.broadcast_to(padk[None], (NW, padw))], axis=1)
    skey = jnp.sort(allk, axis=1)
    src3 = (skey & 16383).reshape(NW, CHUNKS, K)
    dst3 = (skey >> 14).reshape(NW, CHUNKS, K)
    zeros_blk = jnp.zeros((RPT, D), jnp.float32)

    s1 = bng1.reshape(L, 1, D)
    t1 = (convb1 * bng1 + bnb1).reshape(L, 1, D)
    s2 = bns_g.reshape(L, 1, D)
    t2 = (convb2 * bns_g + bns_b).reshape(L, 1, D)

    outs = [x]
    h = x
    for i in range(L):
        parts = _sc_seg_sum(src3, dst3, zeros_blk, h).reshape(NC, N_ACC, D)
        h = _layer_mlp(h, parts, convW1[i], convW2[i],
                       s1[i], t1[i], s2[i], t2[i])
        outs.append(h)

    batch3 = batch.reshape(N_BLKS, 1, ROW_BLK)
    fw_pad = jnp.concatenate(
        [fcW, jnp.zeros((L + 1, D, D - C), jnp.float32)], axis=2)
    sumb = jnp.concatenate(
        [jnp.sum(fcb, axis=0), jnp.zeros((D - C,), jnp.float32)]
    ).reshape(1, D)

    out = _pool_heads(outs[0], outs[1], outs[2], outs[3],
                      batch3, fw_pad, sumb)
    return out[:, :C]


# speedup vs baseline: 8.4144x; 1.3415x over previous
import functools

import jax
import jax.numpy as jnp
from jax import lax
from jax.experimental import pallas as pl
from jax.experimental.pallas import tpu as pltpu
from jax.experimental.pallas import tpu_sc as plsc

N = 10000
E = 320000
D = 128
C = 40
L = 3
G = 128

NC = 2
NS = 16
NW = NC * NS
K = 80
CHUNKS = 128
EPW = CHUNKS * K
E_PAD = NW * EPW
N_ACC = 10240
RPT = N_ACC // NS

ROW_BLK = 400
N_BLKS = N // ROW_BLK


NBUF = 4
ROUNDS = CHUNKS // NBUF


def _sc_body(src_hbm, dst_hbm, zeros_hbm, h_hbm, out_hbm, *sc):
    sidx = sc[0:NBUF]
    didx = sc[NBUF:2 * NBUF]
    rows = sc[2 * NBUF:3 * NBUF]
    acc = sc[3 * NBUF]
    isem = sc[3 * NBUF + 1:4 * NBUF + 1]
    dsem = sc[4 * NBUF + 1:5 * NBUF + 1]
    gsem = sc[5 * NBUF + 1:6 * NBUF + 1]
    ssem = sc[6 * NBUF + 1:7 * NBUF + 1]
    cid = lax.axis_index("c")
    sid = lax.axis_index("s")
    wid = cid * NS + sid
    pltpu.sync_copy(zeros_hbm, acc.at[pl.ds(sid * RPT, RPT)])
    plsc.subcore_barrier()

    for b in range(NBUF):
        pltpu.async_copy(src_hbm.at[wid, b], sidx[b], isem[b])
        pltpu.async_copy(dst_hbm.at[wid, b], didx[b], dsem[b])
    for b in range(NBUF):
        pltpu.make_async_copy(src_hbm.at[wid, b], sidx[b], isem[b]).wait()
        pltpu.async_copy(h_hbm.at[sidx[b]], rows[b], gsem[b])

    def eloop(i, c):
        for b in range(NBUF):
            j = i * NBUF + b
            pltpu.make_async_copy(h_hbm.at[sidx[b]], rows[b], gsem[b]).wait()
            pltpu.make_async_copy(
                dst_hbm.at[wid, j], didx[b], dsem[b]).wait()
            pltpu.async_copy(rows[b], acc.at[didx[b]], ssem[b], add=True)
        @pl.when(i < ROUNDS - 1)
        def _():
            for b in range(NBUF):
                jn = i * NBUF + b + NBUF
                pltpu.async_copy(src_hbm.at[wid, jn], sidx[b], isem[b])
                pltpu.make_async_copy(
                    rows[b], acc.at[didx[b]], ssem[b]).wait()
                pltpu.async_copy(dst_hbm.at[wid, jn], didx[b], dsem[b])
                pltpu.make_async_copy(
                    src_hbm.at[wid, jn], sidx[b], isem[b]).wait()
                pltpu.async_copy(h_hbm.at[sidx[b]], rows[b], gsem[b])
        return c

    lax.fori_loop(0, ROUNDS, eloop, 0)
    for b in range(NBUF):
        pltpu.make_async_copy(rows[b], acc.at[didx[b]], ssem[b]).wait()
    plsc.subcore_barrier()
    pltpu.sync_copy(acc.at[pl.ds(sid * RPT, RPT)],
                    out_hbm.at[pl.ds(cid * N_ACC + sid * RPT, RPT)])


_sc_seg_sum = functools.partial(
    pl.kernel,
    mesh=plsc.VectorSubcoreMesh(core_axis_name="c", subcore_axis_name="s"),
    out_type=jax.ShapeDtypeStruct((NC * N_ACC, D), jnp.float32),
    scratch_types=(
        [pltpu.VMEM((K,), jnp.int32) for _ in range(2 * NBUF)]
        + [pltpu.VMEM((K, D), jnp.float32) for _ in range(NBUF)]
        + [pltpu.VMEM_SHARED((N_ACC, D), jnp.float32)]
        + [pltpu.SemaphoreType.DMA for _ in range(4 * NBUF)]
    ),
)(_sc_body)


def _layer_body(h_ref, p0_ref, p1_ref, w1_ref, w2_ref,
                s1_ref, t1_ref, s2_ref, t2_ref, o_ref):
    z = h_ref[...] + p0_ref[0] + p1_ref[0]
    a = jnp.dot(z, w1_ref[...], preferred_element_type=jnp.float32)
    a = jnp.maximum(a * s1_ref[...] + t1_ref[...], 0.0)
    b = jnp.dot(a, w2_ref[...], preferred_element_type=jnp.float32)
    o_ref[...] = jnp.maximum(b * s2_ref[...] + t2_ref[...], 0.0)


def _layer_mlp(h, parts, w1, w2, s1, t1, s2, t2):
    full = lambda i: (0, 0)
    return pl.pallas_call(
        _layer_body,
        grid=(N_BLKS,),
        in_specs=[
            pl.BlockSpec((ROW_BLK, D), lambda i: (i, 0)),
            pl.BlockSpec((1, ROW_BLK, D), lambda i: (0, i, 0)),
            pl.BlockSpec((1, ROW_BLK, D), lambda i: (1, i, 0)),
            pl.BlockSpec((D, D), full),
            pl.BlockSpec((D, D), full),
            pl.BlockSpec((1, D), full),
            pl.BlockSpec((1, D), full),
            pl.BlockSpec((1, D), full),
            pl.BlockSpec((1, D), full),
        ],
        out_specs=pl.BlockSpec((ROW_BLK, D), lambda i: (i, 0)),
        out_shape=jax.ShapeDtypeStruct((N, D), jnp.float32),
    )(h, parts, parts, w1, w2, s1, t1, s2, t2)


def _pool_body(x_ref, h1_ref, h2_ref, h3_ref, b_ref, fw_ref, sb_ref,
               o_ref, acc_ref):
    i = pl.program_id(0)

    @pl.when(i == 0)
    def _():
        acc_ref[...] = jnp.zeros_like(acc_ref)

    bids = b_ref[0, 0, :]
    m = (lax.broadcasted_iota(jnp.int32, (G, ROW_BLK), 0)
         == bids[None, :]).astype(jnp.float32)
    for k, o in enumerate((x_ref, h1_ref, h2_ref, h3_ref)):
        acc_ref[k] = acc_ref[k] + jnp.dot(
            m, o[...], preferred_element_type=jnp.float32)

    @pl.when(i == N_BLKS - 1)
    def _():
        v = sb_ref[...]
        for k in range(L + 1):
            v = v + jnp.dot(acc_ref[k], fw_ref[k],
                            preferred_element_type=jnp.float32)
        cols = lax.broadcasted_iota(jnp.int32, (G, D), 1)
        valid = cols < C
        vm = jnp.where(valid, v, -1e30)
        mx = jnp.max(vm, axis=1, keepdims=True)
        ex = jnp.where(valid, jnp.exp(vm - mx), 0.0)
        s = jnp.sum(ex, axis=1, keepdims=True)
        o_ref[...] = vm - mx - jnp.log(s)


def _pool_heads(x, h1, h2, h3, batch3, fw_pad, sumb):
    blk = lambda i: (i, 0)
    full = lambda i: (0, 0)
    return pl.pallas_call(
        _pool_body,
        grid=(N_BLKS,),
        in_specs=[
            pl.BlockSpec((ROW_BLK, D), blk),
            pl.BlockSpec((ROW_BLK, D), blk),
            pl.BlockSpec((ROW_BLK, D), blk),
            pl.BlockSpec((ROW_BLK, D), blk),
            pl.BlockSpec((1, 1, ROW_BLK), lambda i: (i, 0, 0)),
            pl.BlockSpec((L + 1, D, D), lambda i: (0, 0, 0)),
            pl.BlockSpec((1, D), full),
        ],
        out_specs=pl.BlockSpec((G, D), full),
        out_shape=jax.ShapeDtypeStruct((G, D), jnp.float32),
        scratch_shapes=[pltpu.VMEM((L + 1, G, D), jnp.float32)],
    )(x, h1, h2, h3, batch3, fw_pad, sumb)


def kernel(x, edge_index, batch, convW1, convb1, bng1, bnb1,
           convW2, convb2, bns_g, bns_b, fcW, fcb):
    src = edge_index[0]
    dst = edge_index[1]
    padw = (E_PAD - E) // NW
    key = (dst << 14) | src
    srows = 128
    padr = (E_PAD - E) // srows
    pad_id = (lax.broadcasted_iota(jnp.int32, (srows, padr), 0) * padr
              + lax.broadcasted_iota(jnp.int32, (srows, padr), 1))
    padk = ((N + pad_id % (N_ACC - N)) << 14) | (pad_id * 41 % N)
    allk = jnp.concatenate(
        [key.reshape(srows, E // srows), padk], axis=1)
    skey = jnp.sort(allk, axis=1)
    src3 = (skey & 16383).reshape(NW, CHUNKS, K)
    dst3 = (skey >> 14).reshape(NW, CHUNKS, K)
    zeros_blk = jnp.zeros((RPT, D), jnp.float32)

    s1 = bng1.reshape(L, 1, D)
    t1 = (convb1 * bng1 + bnb1).reshape(L, 1, D)
    s2 = bns_g.reshape(L, 1, D)
    t2 = (convb2 * bns_g + bns_b).reshape(L, 1, D)

    outs = [x]
    h = x
    for i in range(L):
        parts = _sc_seg_sum(src3, dst3, zeros_blk, h).reshape(NC, N_ACC, D)
        h = _layer_mlp(h, parts, convW1[i], convW2[i],
                       s1[i], t1[i], s2[i], t2[i])
        outs.append(h)

    batch3 = batch.reshape(N_BLKS, 1, ROW_BLK)
    fw_pad = jnp.concatenate(
        [fcW, jnp.zeros((L + 1, D, D - C), jnp.float32)], axis=2)
    sumb = jnp.concatenate(
        [jnp.sum(fcb, axis=0), jnp.zeros((D - C,), jnp.float32)]
    ).reshape(1, D)

    out = _pool_heads(outs[0], outs[1], outs[2], outs[3],
                      batch3, fw_pad, sumb)
    return out[:, :C]


# speedup vs baseline: 8.5542x; 1.0166x over previous
import functools

import jax
import jax.numpy as jnp
from jax import lax
from jax.experimental import pallas as pl
from jax.experimental.pallas import tpu as pltpu
from jax.experimental.pallas import tpu_sc as plsc

N = 10000
E = 320000
D = 128
C = 40
L = 3
G = 128

NC = 2
NS = 16
NW = NC * NS
K = 80
CHUNKS = 128
EPW = CHUNKS * K
E_PAD = NW * EPW
N_ACC = 10240
RPT = N_ACC // NS

ROW_BLK = 400
N_BLKS = N // ROW_BLK


NBUF = 4
ROUNDS = CHUNKS // NBUF


def _sc_body(src_hbm, dst_hbm, zeros_hbm, h_hbm, out_hbm, *sc):
    sidx = sc[0:NBUF]
    didx = sc[NBUF:2 * NBUF]
    rows = sc[2 * NBUF:3 * NBUF]
    acc = sc[3 * NBUF]
    isem = sc[3 * NBUF + 1:4 * NBUF + 1]
    dsem = sc[4 * NBUF + 1:5 * NBUF + 1]
    gsem = sc[5 * NBUF + 1:6 * NBUF + 1]
    ssem = sc[6 * NBUF + 1:7 * NBUF + 1]
    cid = lax.axis_index("c")
    sid = lax.axis_index("s")
    wid = cid * NS + sid
    pltpu.sync_copy(zeros_hbm, acc.at[pl.ds(sid * RPT, RPT)])
    plsc.subcore_barrier()

    for b in range(NBUF):
        pltpu.async_copy(src_hbm.at[wid, b], sidx[b], isem[b])
        pltpu.async_copy(dst_hbm.at[wid, b], didx[b], dsem[b])
    for b in range(NBUF):
        pltpu.make_async_copy(src_hbm.at[wid, b], sidx[b], isem[b]).wait()
        pltpu.async_copy(h_hbm.at[sidx[b]], rows[b], gsem[b])

    def eloop(i, c):
        for b in range(NBUF):
            j = i * NBUF + b
            pltpu.make_async_copy(h_hbm.at[sidx[b]], rows[b], gsem[b]).wait()
            pltpu.make_async_copy(
                dst_hbm.at[wid, j], didx[b], dsem[b]).wait()
            pltpu.async_copy(rows[b], acc.at[didx[b]], ssem[b], add=True)
        @pl.when(i < ROUNDS - 1)
        def _():
            for b in range(NBUF):
                jn = i * NBUF + b + NBUF
                pltpu.async_copy(src_hbm.at[wid, jn], sidx[b], isem[b])
                pltpu.make_async_copy(
                    rows[b], acc.at[didx[b]], ssem[b]).wait()
                pltpu.async_copy(dst_hbm.at[wid, jn], didx[b], dsem[b])
                pltpu.make_async_copy(
                    src_hbm.at[wid, jn], sidx[b], isem[b]).wait()
                pltpu.async_copy(h_hbm.at[sidx[b]], rows[b], gsem[b])
        return c

    lax.fori_loop(0, ROUNDS, eloop, 0)
    for b in range(NBUF):
        pltpu.make_async_copy(rows[b], acc.at[didx[b]], ssem[b]).wait()
    plsc.subcore_barrier()
    pltpu.sync_copy(acc.at[pl.ds(sid * RPT, RPT)],
                    out_hbm.at[pl.ds(cid * N_ACC + sid * RPT, RPT)])


_sc_seg_sum = functools.partial(
    pl.kernel,
    mesh=plsc.VectorSubcoreMesh(core_axis_name="c", subcore_axis_name="s"),
    out_type=jax.ShapeDtypeStruct((NC * N_ACC, D), jnp.float32),
    scratch_types=(
        [pltpu.VMEM((K,), jnp.int32) for _ in range(2 * NBUF)]
        + [pltpu.VMEM((K, D), jnp.float32) for _ in range(NBUF)]
        + [pltpu.VMEM_SHARED((N_ACC, D), jnp.float32)]
        + [pltpu.SemaphoreType.DMA for _ in range(4 * NBUF)]
    ),
)(_sc_body)


def _layer_body(h_ref, p0_ref, p1_ref, w1_ref, w2_ref,
                s1_ref, t1_ref, s2_ref, t2_ref, o_ref):
    z = h_ref[...] + p0_ref[0] + p1_ref[0]
    a = jnp.dot(z, w1_ref[...], preferred_element_type=jnp.float32)
    a = jnp.maximum(a * s1_ref[...] + t1_ref[...], 0.0)
    b = jnp.dot(a, w2_ref[...], preferred_element_type=jnp.float32)
    o_ref[...] = jnp.maximum(b * s2_ref[...] + t2_ref[...], 0.0)


def _layer_mlp(h, parts, w1, w2, s1, t1, s2, t2):
    full = lambda i: (0, 0)
    return pl.pallas_call(
        _layer_body,
        grid=(N_BLKS,),
        in_specs=[
            pl.BlockSpec((ROW_BLK, D), lambda i: (i, 0)),
            pl.BlockSpec((1, ROW_BLK, D), lambda i: (0, i, 0)),
            pl.BlockSpec((1, ROW_BLK, D), lambda i: (1, i, 0)),
            pl.BlockSpec((D, D), full),
            pl.BlockSpec((D, D), full),
            pl.BlockSpec((1, D), full),
            pl.BlockSpec((1, D), full),
            pl.BlockSpec((1, D), full),
            pl.BlockSpec((1, D), full),
        ],
        out_specs=pl.BlockSpec((ROW_BLK, D), lambda i: (i, 0)),
        out_shape=jax.ShapeDtypeStruct((N, D), jnp.float32),
    )(h, parts, parts, w1, w2, s1, t1, s2, t2)


def _pool_body(x_ref, h1_ref, h2_ref, h3_ref, b_ref, fw_ref, sb_ref,
               o_ref, acc_ref):
    i = pl.program_id(0)

    @pl.when(i == 0)
    def _():
        acc_ref[...] = jnp.zeros_like(acc_ref)

    bids = b_ref[0, 0, :]
    m = (lax.broadcasted_iota(jnp.int32, (G, ROW_BLK), 0)
         == bids[None, :]).astype(jnp.float32)
    for k, o in enumerate((x_ref, h1_ref, h2_ref, h3_ref)):
        acc_ref[k] = acc_ref[k] + jnp.dot(
            m, o[...], preferred_element_type=jnp.float32)

    @pl.when(i == N_BLKS - 1)
    def _():
        v = sb_ref[...]
        for k in range(L + 1):
            v = v + jnp.dot(acc_ref[k], fw_ref[k],
                            preferred_element_type=jnp.float32)
        cols = lax.broadcasted_iota(jnp.int32, (G, D), 1)
        valid = cols < C
        vm = jnp.where(valid, v, -1e30)
        mx = jnp.max(vm, axis=1, keepdims=True)
        ex = jnp.where(valid, jnp.exp(vm - mx), 0.0)
        s = jnp.sum(ex, axis=1, keepdims=True)
        o_ref[...] = vm - mx - jnp.log(s)


def _pool_heads(x, h1, h2, h3, batch3, fw_pad, sumb):
    blk = lambda i: (i, 0)
    full = lambda i: (0, 0)
    return pl.pallas_call(
        _pool_body,
        grid=(N_BLKS,),
        in_specs=[
            pl.BlockSpec((ROW_BLK, D), blk),
            pl.BlockSpec((ROW_BLK, D), blk),
            pl.BlockSpec((ROW_BLK, D), blk),
            pl.BlockSpec((ROW_BLK, D), blk),
            pl.BlockSpec((1, 1, ROW_BLK), lambda i: (i, 0, 0)),
            pl.BlockSpec((L + 1, D, D), lambda i: (0, 0, 0)),
            pl.BlockSpec((1, D), full),
        ],
        out_specs=pl.BlockSpec((G, D), full),
        out_shape=jax.ShapeDtypeStruct((G, D), jnp.float32),
        scratch_shapes=[pltpu.VMEM((L + 1, G, D), jnp.float32)],
    )(x, h1, h2, h3, batch3, fw_pad, sumb)


def kernel(x, edge_index, batch, convW1, convb1, bng1, bnb1,
           convW2, convb2, bns_g, bns_b, fcW, fcb):
    src = edge_index[0]
    dst = edge_index[1]
    padw = (E_PAD - E) // NW
    key = (dst << 14) | src
    srows = 512
    padr = (E_PAD - E) // srows
    pad_id = (lax.broadcasted_iota(jnp.int32, (srows, padr), 0) * padr
              + lax.broadcasted_iota(jnp.int32, (srows, padr), 1))
    padk = ((N + pad_id % (N_ACC - N)) << 14) | (pad_id * 41 % N)
    allk = jnp.concatenate(
        [key.reshape(srows, E // srows), padk], axis=1)
    skey = jnp.sort(allk, axis=1)
    src3 = (skey & 16383).reshape(NW, CHUNKS, K)
    dst3 = (skey >> 14).reshape(NW, CHUNKS, K)
    zeros_blk = jnp.zeros((RPT, D), jnp.float32)

    s1 = bng1.reshape(L, 1, D)
    t1 = (convb1 * bng1 + bnb1).reshape(L, 1, D)
    s2 = bns_g.reshape(L, 1, D)
    t2 = (convb2 * bns_g + bns_b).reshape(L, 1, D)

    outs = [x]
    h = x
    for i in range(L):
        parts = _sc_seg_sum(src3, dst3, zeros_blk, h).reshape(NC, N_ACC, D)
        h = _layer_mlp(h, parts, convW1[i], convW2[i],
                       s1[i], t1[i], s2[i], t2[i])
        outs.append(h)

    batch3 = batch.reshape(N_BLKS, 1, ROW_BLK)
    fw_pad = jnp.concatenate(
        [fcW, jnp.zeros((L + 1, D, D - C), jnp.float32)], axis=2)
    sumb = jnp.concatenate(
        [jnp.sum(fcb, axis=0), jnp.zeros((D - C,), jnp.float32)]
    ).reshape(1, D)

    out = _pool_heads(outs[0], outs[1], outs[2], outs[3],
                      batch3, fw_pad, sumb)
    return out[:, :C]
